# baseline (speedup 1.0000x reference)
import functools

import jax
import jax.numpy as jnp
from jax import lax
from jax.experimental import pallas as pl
from jax.experimental.pallas import tpu as pltpu
from jax.experimental.pallas import tpu_sc as plsc

B = 32
N_OBJ = 16
D = 32
N_NEURAL = 50000
N_ATOMS = 100000

TBLK = 4096
NJ = (N_ATOMS + TBLK - 1) // TBLK
NEURAL_HI = N_NEURAL + 2
NBLK_NEURAL = (NEURAL_HI + TBLK - 1) // TBLK
WPAD = NBLK_NEURAL * TBLK
NBINS = NJ * TBLK

N_BASE_PAD = 5120
PER_SC = N_BASE_PAD // 2
PER_SUB = PER_SC // 16
CHUNK = PER_SUB // 2
ZSPAN = NBINS // 16


def _hist_body(idx_hbm, out_hbm, idx_a, idx_b, ones_v, buf_v, hist_sh):
    cid = lax.axis_index("c")
    sid = lax.axis_index("s")

    for i in range(CHUNK // 16):
        ones_v[pl.ds(i * 16, 16)] = jnp.full((16,), 1.0, jnp.float32)

    def _zero(i, carry):
        buf_v[pl.ds(i * 16, 16)] = jnp.zeros((16,), jnp.float32)
        return carry

    lax.fori_loop(0, ZSPAN // 16, _zero, 0)

    zbase = sid * ZSPAN
    pltpu.sync_copy(buf_v, hist_sh.at[pl.ds(zbase, ZSPAN)])
    plsc.subcore_barrier()

    base = cid * PER_SC + sid * PER_SUB
    pltpu.sync_copy(idx_hbm.at[pl.ds(base, CHUNK)], idx_a)
    pltpu.sync_copy(idx_hbm.at[pl.ds(base + CHUNK, CHUNK)], idx_b)
    pltpu.sync_copy(ones_v, hist_sh.at[idx_a], add=True)
    pltpu.sync_copy(ones_v, hist_sh.at[idx_b], add=True)
    plsc.subcore_barrier()

    pltpu.sync_copy(hist_sh.at[pl.ds(zbase, ZSPAN)],
                    out_hbm.at[cid, pl.ds(zbase, ZSPAN)])


@functools.cache
def _get_hist_kernel():
    return pl.kernel(
        _hist_body,
        out_type=jax.ShapeDtypeStruct((2, NBINS), jnp.float32),
        mesh=plsc.VectorSubcoreMesh(core_axis_name="c", subcore_axis_name="s"),
        scratch_types=[
            pltpu.VMEM((CHUNK,), jnp.int32),
            pltpu.VMEM((CHUNK,), jnp.int32),
            pltpu.VMEM((CHUNK,), jnp.float32),
            pltpu.VMEM((ZSPAN,), jnp.float32),
            pltpu.VMEM_SHARED((NBINS,), jnp.float32),
        ],
    )


LASTW = N_NEURAL - (NBLK_NEURAL - 1) * TBLK


def _tc_body(zf_ref, w_hbm, obj_ref, eye_ref, cnt_ref, out_ref,
             wv_ref, carry_ref, sem):
    j = pl.program_id(0)
    csum = cnt_ref[0:1, :] + cnt_ref[1:2, :]
    col = j * TBLK + lax.broadcasted_iota(jnp.int32, (1, TBLK), 1)

    @pl.when(j < NBLK_NEURAL)
    def _():
        @pl.when(j < NBLK_NEURAL - 1)
        def _():
            cp = pltpu.make_async_copy(
                w_hbm.at[pl.ds(j * TBLK, TBLK)], wv_ref, sem)
            cp.start()
            cp.wait()

        @pl.when(j == NBLK_NEURAL - 1)
        def _():
            cp = pltpu.make_async_copy(
                w_hbm.at[pl.ds(j * TBLK, LASTW)],
                wv_ref.at[pl.ds(0, LASTW)], sem)
            cp.start()
            cp.wait()

        wnat = wv_ref[...].astype(jnp.bfloat16)
        wtb = lax.dot_general(eye_ref[...], wnat, (((1,), (1,)), ((), ())),
                              preferred_element_type=jnp.float32
                              ).astype(jnp.bfloat16)
        objr = obj_ref[...].astype(jnp.bfloat16)
        parts = [jnp.where(objr == jnp.bfloat16(o), wtb, jnp.bfloat16(0.0))
                 for o in range(N_OBJ)]
        u = jnp.concatenate(parts, axis=0)
        acc = jnp.dot(zf_ref[...], u, preferred_element_type=jnp.float32)
        vals = jax.nn.sigmoid(acc)
        prev = carry_ref[...]
        carry_ref[...] = vals[:, TBLK - 128:]
        sh = pltpu.roll(vals, 2, 1)
        rc = pltpu.roll(prev, 2, 1)
        head = jnp.where(lax.broadcasted_iota(jnp.int32, (1, 128), 1) < 2,
                         rc, sh[:, :128])
        vfull = jnp.concatenate([head, sh[:, 128:]], axis=1)
        neural = (col >= 2) & (col < NEURAL_HI)
        out = jnp.where(neural, vfull, 0.0) + csum
        out_ref[...] = jnp.where(col == 1, 1.0, out)

    @pl.when(j >= NBLK_NEURAL)
    def _():
        out_ref[...] = jnp.broadcast_to(csum, (B, TBLK))


def kernel(Z, atom_idx, obj_idx, w, base_idx):
    del atom_idx
    zflat = Z.reshape(B, N_OBJ * D).astype(jnp.bfloat16)
    obj2 = obj_idx.reshape(1, N_NEURAL)
    base_p = jnp.pad(base_idx, (0, N_BASE_PAD - base_idx.shape[0]),
                     constant_values=N_ATOMS)

    counts = _get_hist_kernel()(base_p)

    return pl.pallas_call(
        _tc_body,
        grid=(NJ,),
        in_specs=[
            pl.BlockSpec((B, N_OBJ * D), lambda j: (0, 0)),
            pl.BlockSpec(memory_space=pl.ANY),
            pl.BlockSpec((1, TBLK), lambda j: (0, jnp.minimum(j, NBLK_NEURAL - 1))),
            pl.BlockSpec((D, D), lambda j: (0, 0)),
            pl.BlockSpec((2, TBLK), lambda j: (0, j)),
        ],
        out_specs=pl.BlockSpec((B, TBLK), lambda j: (0, j)),
        out_shape=jax.ShapeDtypeStruct((B, N_ATOMS), jnp.float32),
        scratch_shapes=[pltpu.VMEM((TBLK, D), jnp.float32),
                        pltpu.VMEM((B, 128), jnp.float32),
                        pltpu.SemaphoreType.DMA],
    )(zflat, w, obj2, jnp.eye(D, dtype=jnp.bfloat16), counts)


# speedup vs baseline: 1.7316x; 1.7316x over previous
import functools

import jax
import jax.numpy as jnp
from jax import lax
from jax.experimental import pallas as pl
from jax.experimental.pallas import tpu as pltpu
from jax.experimental.pallas import tpu_sc as plsc

B = 32
N_OBJ = 16
D = 32
N_NEURAL = 50000
N_ATOMS = 100000

TBLK = 4096
NJ = (N_ATOMS + TBLK - 1) // TBLK
NEURAL_HI = N_NEURAL + 2
NBLK_NEURAL = (NEURAL_HI + TBLK - 1) // TBLK
WPAD = NBLK_NEURAL * TBLK
NBINS = NJ * TBLK

N_BASE_PAD = 5120
PER_SC = N_BASE_PAD // 2
PER_SUB = PER_SC // 16
CHUNK = PER_SUB // 2
ZSPAN = NBINS // 16


def _hist_body(idx_hbm, out_hbm, idx_a, idx_b, ones_v, buf_v, hist_sh):
    cid = lax.axis_index("c")
    sid = lax.axis_index("s")

    for i in range(CHUNK // 16):
        ones_v[pl.ds(i * 16, 16)] = jnp.full((16,), 1.0, jnp.float32)

    def _zero(i, carry):
        buf_v[pl.ds(i * 16, 16)] = jnp.zeros((16,), jnp.float32)
        return carry

    lax.fori_loop(0, ZSPAN // 16, _zero, 0)

    zbase = sid * ZSPAN
    pltpu.sync_copy(buf_v, hist_sh.at[pl.ds(zbase, ZSPAN)])
    plsc.subcore_barrier()

    base = cid * PER_SC + sid * PER_SUB
    pltpu.sync_copy(idx_hbm.at[pl.ds(base, CHUNK)], idx_a)
    pltpu.sync_copy(idx_hbm.at[pl.ds(base + CHUNK, CHUNK)], idx_b)
    pltpu.sync_copy(ones_v, hist_sh.at[idx_a], add=True)
    pltpu.sync_copy(ones_v, hist_sh.at[idx_b], add=True)
    plsc.subcore_barrier()

    pltpu.sync_copy(hist_sh.at[pl.ds(zbase, ZSPAN)],
                    out_hbm.at[cid, pl.ds(zbase, ZSPAN)])


@functools.cache
def _get_hist_kernel():
    return pl.kernel(
        _hist_body,
        out_type=jax.ShapeDtypeStruct((2, NBINS), jnp.float32),
        mesh=plsc.VectorSubcoreMesh(core_axis_name="c", subcore_axis_name="s"),
        scratch_types=[
            pltpu.VMEM((CHUNK,), jnp.int32),
            pltpu.VMEM((CHUNK,), jnp.int32),
            pltpu.VMEM((CHUNK,), jnp.float32),
            pltpu.VMEM((ZSPAN,), jnp.float32),
            pltpu.VMEM_SHARED((NBINS,), jnp.float32),
        ],
    )


def _tc_body(zf_ref, w_ref, obj_ref, cnt_ref, out_ref, carry_ref):
    j = pl.program_id(0)
    csum = cnt_ref[0:1, :] + cnt_ref[1:2, :]
    col = j * TBLK + lax.broadcasted_iota(jnp.int32, (1, TBLK), 1)

    @pl.when(j < NBLK_NEURAL)
    def _():
        wtb = w_ref[...]
        objr = obj_ref[...].astype(jnp.bfloat16)
        parts = [jnp.where(objr == jnp.bfloat16(o), wtb, jnp.bfloat16(0.0))
                 for o in range(N_OBJ)]
        u = jnp.concatenate(parts, axis=0)
        acc = jnp.dot(zf_ref[...], u, preferred_element_type=jnp.float32)
        vals = jax.nn.sigmoid(acc)
        prev = carry_ref[...]
        carry_ref[...] = vals[:, TBLK - 128:]
        sh = pltpu.roll(vals, 2, 1)
        rc = pltpu.roll(prev, 2, 1)
        head = jnp.where(lax.broadcasted_iota(jnp.int32, (1, 128), 1) < 2,
                         rc, sh[:, :128])
        vfull = jnp.concatenate([head, sh[:, 128:]], axis=1)
        neural = (col >= 2) & (col < NEURAL_HI)
        out = jnp.where(neural, vfull, 0.0) + csum
        out_ref[...] = jnp.where(col == 1, 1.0, out)

    @pl.when(j >= NBLK_NEURAL)
    def _():
        out_ref[...] = jnp.broadcast_to(csum, (B, TBLK))


def kernel(Z, atom_idx, obj_idx, w, base_idx):
    del atom_idx
    zflat = Z.reshape(B, N_OBJ * D).astype(jnp.bfloat16)
    wt = w.astype(jnp.bfloat16).T
    obj2 = obj_idx.reshape(1, N_NEURAL)
    base_p = jnp.pad(base_idx, (0, N_BASE_PAD - base_idx.shape[0]),
                     constant_values=N_ATOMS)

    counts = _get_hist_kernel()(base_p)

    return pl.pallas_call(
        _tc_body,
        grid=(NJ,),
        in_specs=[
            pl.BlockSpec((B, N_OBJ * D), lambda j: (0, 0)),
            pl.BlockSpec((B, TBLK), lambda j: (0, jnp.minimum(j, NBLK_NEURAL - 1))),
            pl.BlockSpec((1, TBLK), lambda j: (0, jnp.minimum(j, NBLK_NEURAL - 1))),
            pl.BlockSpec((2, TBLK), lambda j: (0, j)),
        ],
        out_specs=pl.BlockSpec((B, TBLK), lambda j: (0, j)),
        out_shape=jax.ShapeDtypeStruct((B, N_ATOMS), jnp.float32),
        scratch_shapes=[pltpu.VMEM((B, 128), jnp.float32)],
    )(zflat, wt, obj2, counts)


# speedup vs baseline: 1.8350x; 1.0598x over previous
import functools

import jax
import jax.numpy as jnp
from jax import lax
from jax.experimental import pallas as pl
from jax.experimental.pallas import tpu as pltpu
from jax.experimental.pallas import tpu_sc as plsc

B = 32
N_OBJ = 16
D = 32
N_NEURAL = 50000
N_ATOMS = 100000

TBLK = 8192
NJ = (N_ATOMS + TBLK - 1) // TBLK
NEURAL_HI = N_NEURAL + 2
NBLK_NEURAL = (NEURAL_HI + TBLK - 1) // TBLK
WPAD = NBLK_NEURAL * TBLK
NBINS = NJ * TBLK

N_BASE_PAD = 5120
PER_SC = N_BASE_PAD // 2
PER_SUB = PER_SC // 16
CHUNK = PER_SUB // 2
ZSPAN = NBINS // 16


def _hist_body(idx_hbm, out_hbm, idx_a, idx_b, ones_v, buf_v, hist_sh):
    cid = lax.axis_index("c")
    sid = lax.axis_index("s")

    for i in range(CHUNK // 16):
        ones_v[pl.ds(i * 16, 16)] = jnp.full((16,), 1.0, jnp.float32)

    def _zero(i, carry):
        buf_v[pl.ds(i * 16, 16)] = jnp.zeros((16,), jnp.float32)
        return carry

    lax.fori_loop(0, ZSPAN // 16, _zero, 0)

    zbase = sid * ZSPAN
    pltpu.sync_copy(buf_v, hist_sh.at[pl.ds(zbase, ZSPAN)])
    plsc.subcore_barrier()

    base = cid * PER_SC + sid * PER_SUB
    pltpu.sync_copy(idx_hbm.at[pl.ds(base, CHUNK)], idx_a)
    pltpu.sync_copy(idx_hbm.at[pl.ds(base + CHUNK, CHUNK)], idx_b)
    pltpu.sync_copy(ones_v, hist_sh.at[idx_a], add=True)
    pltpu.sync_copy(ones_v, hist_sh.at[idx_b], add=True)
    plsc.subcore_barrier()

    pltpu.sync_copy(hist_sh.at[pl.ds(zbase, ZSPAN)],
                    out_hbm.at[cid, pl.ds(zbase, ZSPAN)])


@functools.cache
def _get_hist_kernel():
    return pl.kernel(
        _hist_body,
        out_type=jax.ShapeDtypeStruct((2, NBINS), jnp.float32),
        mesh=plsc.VectorSubcoreMesh(core_axis_name="c", subcore_axis_name="s"),
        scratch_types=[
            pltpu.VMEM((CHUNK,), jnp.int32),
            pltpu.VMEM((CHUNK,), jnp.int32),
            pltpu.VMEM((CHUNK,), jnp.float32),
            pltpu.VMEM((ZSPAN,), jnp.float32),
            pltpu.VMEM_SHARED((NBINS,), jnp.float32),
        ],
    )


def _tc_body(zf_ref, w_ref, obj_ref, cnt_ref, out_ref, carry_ref):
    j = pl.program_id(0)
    csum = cnt_ref[0:1, :] + cnt_ref[1:2, :]
    col = j * TBLK + lax.broadcasted_iota(jnp.int32, (1, TBLK), 1)

    @pl.when(j < NBLK_NEURAL)
    def _():
        wtb = w_ref[...]
        objr = obj_ref[...].astype(jnp.bfloat16)
        parts = [jnp.where(objr == jnp.bfloat16(o), wtb, jnp.bfloat16(0.0))
                 for o in range(N_OBJ)]
        u = jnp.concatenate(parts, axis=0)
        acc = jnp.dot(zf_ref[...], u, preferred_element_type=jnp.float32)
        vals = jax.nn.sigmoid(acc)
        prev = carry_ref[...]
        carry_ref[...] = vals[:, TBLK - 128:]
        sh = pltpu.roll(vals, 2, 1)
        rc = pltpu.roll(prev, 2, 1)
        head = jnp.where(lax.broadcasted_iota(jnp.int32, (1, 128), 1) < 2,
                         rc, sh[:, :128])
        vfull = jnp.concatenate([head, sh[:, 128:]], axis=1)
        neural = (col >= 2) & (col < NEURAL_HI)
        out = jnp.where(neural, vfull, 0.0) + csum
        out_ref[...] = jnp.where(col == 1, 1.0, out)

    @pl.when(j >= NBLK_NEURAL)
    def _():
        out_ref[...] = jnp.broadcast_to(csum, (B, TBLK))


def kernel(Z, atom_idx, obj_idx, w, base_idx):
    del atom_idx
    zflat = Z.reshape(B, N_OBJ * D).astype(jnp.bfloat16)
    wt = w.astype(jnp.bfloat16).T
    obj2 = obj_idx.reshape(1, N_NEURAL)
    base_p = jnp.pad(base_idx, (0, N_BASE_PAD - base_idx.shape[0]),
                     constant_values=N_ATOMS)

    counts = _get_hist_kernel()(base_p)

    return pl.pallas_call(
        _tc_body,
        grid=(NJ,),
        in_specs=[
            pl.BlockSpec((B, N_OBJ * D), lambda j: (0, 0)),
            pl.BlockSpec((B, TBLK), lambda j: (0, jnp.minimum(j, NBLK_NEURAL - 1))),
            pl.BlockSpec((1, TBLK), lambda j: (0, jnp.minimum(j, NBLK_NEURAL - 1))),
            pl.BlockSpec((2, TBLK), lambda j: (0, j)),
        ],
        out_specs=pl.BlockSpec((B, TBLK), lambda j: (0, j)),
        out_shape=jax.ShapeDtypeStruct((B, N_ATOMS), jnp.float32),
        scratch_shapes=[pltpu.VMEM((B, 128), jnp.float32)],
    )(zflat, wt, obj2, counts)


# speedup vs baseline: 1.9790x; 1.0784x over previous
import functools

import jax
import jax.numpy as jnp
from jax import lax
from jax.experimental import pallas as pl
from jax.experimental.pallas import tpu as pltpu
from jax.experimental.pallas import tpu_sc as plsc

B = 32
N_OBJ = 16
D = 32
N_NEURAL = 50000
N_ATOMS = 100000

TBLK = 8192
NJ = (N_ATOMS + TBLK - 1) // TBLK
NEURAL_HI = N_NEURAL + 2
NBLK_NEURAL = (NEURAL_HI + TBLK - 1) // TBLK
WPAD = NBLK_NEURAL * TBLK
NBINS = NJ * TBLK

N_BASE_PAD = 5120
PER_SC = N_BASE_PAD // 2
PER_SUB = PER_SC // 16
CHUNK = PER_SUB // 2
ZSPAN = NBINS // 16


def _hist_body(idx_hbm, out_hbm, idx_a, idx_b, ones_v, buf_v, hist_sh):
    cid = lax.axis_index("c")
    sid = lax.axis_index("s")

    for i in range(CHUNK // 16):
        ones_v[pl.ds(i * 16, 16)] = jnp.full((16,), 1.0, jnp.float32)

    def _zero(i, carry):
        buf_v[pl.ds(i * 16, 16)] = jnp.zeros((16,), jnp.float32)
        return carry

    lax.fori_loop(0, ZSPAN // 16, _zero, 0)

    zbase = sid * ZSPAN
    pltpu.sync_copy(buf_v, hist_sh.at[pl.ds(zbase, ZSPAN)])
    plsc.subcore_barrier()

    base = cid * PER_SC + sid * PER_SUB
    pltpu.sync_copy(idx_hbm.at[pl.ds(base, CHUNK)], idx_a)
    pltpu.sync_copy(idx_hbm.at[pl.ds(base + CHUNK, CHUNK)], idx_b)
    pltpu.sync_copy(ones_v, hist_sh.at[idx_a], add=True)
    pltpu.sync_copy(ones_v, hist_sh.at[idx_b], add=True)
    plsc.subcore_barrier()

    pltpu.sync_copy(hist_sh.at[pl.ds(zbase, ZSPAN)],
                    out_hbm.at[cid, pl.ds(zbase, ZSPAN)])


@functools.cache
def _get_hist_kernel():
    return pl.kernel(
        _hist_body,
        out_type=jax.ShapeDtypeStruct((2, NBINS), jnp.float32),
        mesh=plsc.VectorSubcoreMesh(core_axis_name="c", subcore_axis_name="s"),
        scratch_types=[
            pltpu.VMEM((CHUNK,), jnp.int32),
            pltpu.VMEM((CHUNK,), jnp.int32),
            pltpu.VMEM((CHUNK,), jnp.float32),
            pltpu.VMEM((ZSPAN,), jnp.float32),
            pltpu.VMEM_SHARED((NBINS,), jnp.float32),
        ],
    )


def _tc_body(zf_ref, w_ref, obj_ref, cnt_ref, out_ref, carry_ref):
    j = pl.program_id(0)
    csum = cnt_ref[0:1, :] + cnt_ref[1:2, :]
    col = j * TBLK + lax.broadcasted_iota(jnp.int32, (1, TBLK), 1)

    @pl.when(j < NBLK_NEURAL)
    def _():
        wtb = w_ref[...]
        objr = obj_ref[...].astype(jnp.bfloat16)
        ohs = [jnp.where(objr == jnp.bfloat16(o), jnp.bfloat16(1.0),
                         jnp.bfloat16(0.0)) for o in range(N_OBJ)]
        parts = [wtb * oh for oh in ohs]
        u = jnp.concatenate(parts, axis=0)
        acc = jnp.dot(zf_ref[...], u, preferred_element_type=jnp.float32)
        vals = jax.nn.sigmoid(acc)
        prev = carry_ref[...]
        carry_ref[...] = vals[:, TBLK - 128:]
        sh = pltpu.roll(vals, 2, 1)
        rc = pltpu.roll(prev, 2, 1)
        head = jnp.where(lax.broadcasted_iota(jnp.int32, (1, 128), 1) < 2,
                         rc, sh[:, :128])
        vfull = jnp.concatenate([head, sh[:, 128:]], axis=1)
        neural = (col >= 2) & (col < NEURAL_HI)
        out = jnp.where(neural, vfull, 0.0) + csum
        out_ref[...] = jnp.where(col == 1, 1.0, out)

    @pl.when(j >= NBLK_NEURAL)
    def _():
        out_ref[...] = jnp.broadcast_to(csum, (B, TBLK))


def kernel(Z, atom_idx, obj_idx, w, base_idx):
    del atom_idx
    zflat = Z.reshape(B, N_OBJ * D).astype(jnp.bfloat16)
    wt = w.astype(jnp.bfloat16).T
    obj2 = obj_idx.reshape(1, N_NEURAL)
    base_p = jnp.pad(base_idx, (0, N_BASE_PAD - base_idx.shape[0]),
                     constant_values=N_ATOMS)

    counts = _get_hist_kernel()(base_p)

    return pl.pallas_call(
        _tc_body,
        grid=(NJ,),
        in_specs=[
            pl.BlockSpec((B, N_OBJ * D), lambda j: (0, 0)),
            pl.BlockSpec((B, TBLK), lambda j: (0, jnp.minimum(j, NBLK_NEURAL - 1))),
            pl.BlockSpec((1, TBLK), lambda j: (0, jnp.minimum(j, NBLK_NEURAL - 1))),
            pl.BlockSpec((2, TBLK), lambda j: (0, j)),
        ],
        out_specs=pl.BlockSpec((B, TBLK), lambda j: (0, j)),
        out_shape=jax.ShapeDtypeStruct((B, N_ATOMS), jnp.float32),
        scratch_shapes=[pltpu.VMEM((B, 128), jnp.float32)],
    )(zflat, wt, obj2, counts)


# speedup vs baseline: 2.0067x; 1.0140x over previous
import functools

import jax
import jax.numpy as jnp
from jax import lax
from jax.experimental import pallas as pl
from jax.experimental.pallas import tpu as pltpu
from jax.experimental.pallas import tpu_sc as plsc

B = 32
N_OBJ = 16
D = 32
N_NEURAL = 50000
N_ATOMS = 100000

TBLK = 8192
NJ = (N_ATOMS + TBLK - 1) // TBLK
NEURAL_HI = N_NEURAL + 2
NBLK_NEURAL = (NEURAL_HI + TBLK - 1) // TBLK
WPAD = NBLK_NEURAL * TBLK
NBINS = NJ * TBLK

N_BASE = 5000
PER_SUB = 160
CHUNK = 80
TAIL = N_BASE - 31 * PER_SUB
ZSPAN = NBINS // 16


def _hist_body(idx_hbm, out_hbm, idx_a, idx_b, idx_c, ones_v, ones_t,
               buf_v, hist_sh):
    cid = lax.axis_index("c")
    sid = lax.axis_index("s")
    wid = cid * 16 + sid

    for i in range(CHUNK // 16):
        ones_v[pl.ds(i * 16, 16)] = jnp.full((16,), 1.0, jnp.float32)
    for i in range(TAIL // 16):
        ones_t[pl.ds(i * 16, 16)] = jnp.full((16,), 1.0, jnp.float32)

    def _zero(i, carry):
        buf_v[pl.ds(i * 16, 16)] = jnp.zeros((16,), jnp.float32)
        return carry

    lax.fori_loop(0, ZSPAN // 16, _zero, 0)

    zbase = sid * ZSPAN
    pltpu.sync_copy(buf_v, hist_sh.at[pl.ds(zbase, ZSPAN)])
    plsc.subcore_barrier()

    base = wid * PER_SUB

    @pl.when(wid < 31)
    def _():
        pltpu.sync_copy(idx_hbm.at[pl.ds(base, CHUNK)], idx_a)
        pltpu.sync_copy(idx_hbm.at[pl.ds(base + CHUNK, CHUNK)], idx_b)
        pltpu.sync_copy(ones_v, hist_sh.at[idx_a], add=True)
        pltpu.sync_copy(ones_v, hist_sh.at[idx_b], add=True)

    @pl.when(wid == 31)
    def _():
        pltpu.sync_copy(idx_hbm.at[pl.ds(31 * PER_SUB, TAIL)], idx_c)
        pltpu.sync_copy(ones_t, hist_sh.at[idx_c], add=True)

    plsc.subcore_barrier()

    pltpu.sync_copy(hist_sh.at[pl.ds(zbase, ZSPAN)],
                    out_hbm.at[cid, pl.ds(zbase, ZSPAN)])


@functools.cache
def _get_hist_kernel():
    return pl.kernel(
        _hist_body,
        out_type=jax.ShapeDtypeStruct((2, NBINS), jnp.float32),
        mesh=plsc.VectorSubcoreMesh(core_axis_name="c", subcore_axis_name="s"),
        scratch_types=[
            pltpu.VMEM((CHUNK,), jnp.int32),
            pltpu.VMEM((CHUNK,), jnp.int32),
            pltpu.VMEM((TAIL,), jnp.int32),
            pltpu.VMEM((CHUNK,), jnp.float32),
            pltpu.VMEM((TAIL,), jnp.float32),
            pltpu.VMEM((ZSPAN,), jnp.float32),
            pltpu.VMEM_SHARED((NBINS,), jnp.float32),
        ],
    )


def _tc_body(zf_ref, w_ref, obj_ref, cnt_ref, out_ref, carry_ref):
    j = pl.program_id(0)
    csum = cnt_ref[0:1, :] + cnt_ref[1:2, :]
    col = j * TBLK + lax.broadcasted_iota(jnp.int32, (1, TBLK), 1)

    @pl.when(j < NBLK_NEURAL)
    def _():
        wtb = w_ref[...]
        objr = obj_ref[...]
        ohs = [jnp.where(objr == jnp.bfloat16(o), jnp.bfloat16(1.0),
                         jnp.bfloat16(0.0)) for o in range(N_OBJ)]
        parts = [wtb * oh for oh in ohs]
        u = jnp.concatenate(parts, axis=0)
        acc = jnp.dot(zf_ref[...], u, preferred_element_type=jnp.float32)
        vals = jax.nn.sigmoid(acc)
        prev = carry_ref[...]
        carry_ref[...] = vals[:, TBLK - 128:]
        sh = pltpu.roll(vals, 2, 1)
        rc = pltpu.roll(prev, 2, 1)
        head = jnp.where(lax.broadcasted_iota(jnp.int32, (1, 128), 1) < 2,
                         rc, sh[:, :128])
        vfull = jnp.concatenate([head, sh[:, 128:]], axis=1)
        neural = (col >= 2) & (col < NEURAL_HI)
        out = jnp.where(neural, vfull, 0.0) + csum
        out_ref[...] = jnp.where(col == 1, 1.0, out)

    @pl.when(j >= NBLK_NEURAL)
    def _():
        out_ref[...] = jnp.broadcast_to(csum, (B, TBLK))


def kernel(Z, atom_idx, obj_idx, w, base_idx):
    del atom_idx
    zflat = Z.reshape(B, N_OBJ * D).astype(jnp.bfloat16)
    wt = w.astype(jnp.bfloat16).T
    obj2 = obj_idx.astype(jnp.bfloat16).reshape(1, N_NEURAL)

    counts = _get_hist_kernel()(base_idx)

    return pl.pallas_call(
        _tc_body,
        grid=(NJ,),
        in_specs=[
            pl.BlockSpec((B, N_OBJ * D), lambda j: (0, 0)),
            pl.BlockSpec((B, TBLK), lambda j: (0, jnp.minimum(j, NBLK_NEURAL - 1))),
            pl.BlockSpec((1, TBLK), lambda j: (0, jnp.minimum(j, NBLK_NEURAL - 1))),
            pl.BlockSpec((2, TBLK), lambda j: (0, j)),
        ],
        out_specs=pl.BlockSpec((B, TBLK), lambda j: (0, j)),
        out_shape=jax.ShapeDtypeStruct((B, N_ATOMS), jnp.float32),
        scratch_shapes=[pltpu.VMEM((B, 128), jnp.float32)],
    )(zflat, wt, obj2, counts)


# speedup vs baseline: 2.0078x; 1.0006x over previous
import functools

import jax
import jax.numpy as jnp
from jax import lax
from jax.experimental import pallas as pl
from jax.experimental.pallas import tpu as pltpu
from jax.experimental.pallas import tpu_sc as plsc

B = 32
N_OBJ = 16
D = 32
N_NEURAL = 50000
N_ATOMS = 100000

TBLK = 8192
NJ = (N_ATOMS + TBLK - 1) // TBLK
NEURAL_HI = N_NEURAL + 2
NBLK_NEURAL = (NEURAL_HI + TBLK - 1) // TBLK
WPAD = NBLK_NEURAL * TBLK
NBINS = NJ * TBLK

N_BASE = 5000
PER_SUB = 160
CHUNK = 80
TAIL = N_BASE - 31 * PER_SUB
ZSPAN = NBINS // 16


def _hist_body(idx_hbm, out_hbm, idx_a, idx_b, idx_c, ones_v,
               buf_v, hist_sh):
    cid = lax.axis_index("c")
    sid = lax.axis_index("s")
    wid = cid * 16 + sid

    for i in range(CHUNK // 16):
        ones_v[pl.ds(i * 16, 16)] = jnp.full((16,), 1.0, jnp.float32)

    def _zero(i, carry):
        buf_v[pl.ds(i * 16, 16)] = jnp.zeros((16,), jnp.float32)
        return carry

    lax.fori_loop(0, ZSPAN // 16, _zero, 0)

    zbase = sid * ZSPAN
    pltpu.sync_copy(buf_v, hist_sh.at[pl.ds(zbase, ZSPAN)])
    plsc.subcore_barrier()

    base = wid * PER_SUB

    @pl.when(wid < 31)
    def _():
        pltpu.sync_copy(idx_hbm.at[pl.ds(base, CHUNK)], idx_a)
        pltpu.sync_copy(idx_hbm.at[pl.ds(base + CHUNK, CHUNK)], idx_b)
        pltpu.sync_copy(ones_v, hist_sh.at[idx_a], add=True)
        pltpu.sync_copy(ones_v, hist_sh.at[idx_b], add=True)

    @pl.when(wid == 31)
    def _():
        pltpu.sync_copy(idx_hbm.at[pl.ds(31 * PER_SUB, TAIL)], idx_c)
        pltpu.sync_copy(ones_v.at[pl.ds(0, TAIL)], hist_sh.at[idx_c], add=True)

    plsc.subcore_barrier()

    pltpu.sync_copy(hist_sh.at[pl.ds(zbase, ZSPAN)],
                    out_hbm.at[cid, pl.ds(zbase, ZSPAN)])


@functools.cache
def _get_hist_kernel():
    return pl.kernel(
        _hist_body,
        out_type=jax.ShapeDtypeStruct((2, NBINS), jnp.float32),
        mesh=plsc.VectorSubcoreMesh(core_axis_name="c", subcore_axis_name="s"),
        scratch_types=[
            pltpu.VMEM((CHUNK,), jnp.int32),
            pltpu.VMEM((CHUNK,), jnp.int32),
            pltpu.VMEM((TAIL,), jnp.int32),
            pltpu.VMEM((CHUNK,), jnp.float32),
            pltpu.VMEM((ZSPAN,), jnp.float32),
            pltpu.VMEM_SHARED((NBINS,), jnp.float32),
        ],
    )


def _tc_body(zf_ref, w_ref, obj_ref, cnt_ref, out_ref, carry_ref):
    j = pl.program_id(0)
    csum = cnt_ref[0:1, :] + cnt_ref[1:2, :]
    col = j * TBLK + lax.broadcasted_iota(jnp.int32, (1, TBLK), 1)

    @pl.when(j < NBLK_NEURAL)
    def _():
        wtb = w_ref[...]
        objr = obj_ref[...]
        ohs = [jnp.where(objr == jnp.bfloat16(o), jnp.bfloat16(1.0),
                         jnp.bfloat16(0.0)) for o in range(N_OBJ)]
        parts = [wtb * oh for oh in ohs]
        u = jnp.concatenate(parts, axis=0)
        acc = jnp.dot(zf_ref[...], u, preferred_element_type=jnp.float32)
        vals = jax.nn.sigmoid(acc)
        prev = carry_ref[...]
        carry_ref[...] = vals[:, TBLK - 128:]
        sh = pltpu.roll(vals, 2, 1)
        rc = pltpu.roll(prev, 2, 1)
        head = jnp.where(lax.broadcasted_iota(jnp.int32, (1, 128), 1) < 2,
                         rc, sh[:, :128])
        vfull = jnp.concatenate([head, sh[:, 128:]], axis=1)
        neural = (col >= 2) & (col < NEURAL_HI)
        out = jnp.where(neural, vfull, 0.0) + csum
        out_ref[...] = jnp.where(col == 1, 1.0, out)

    @pl.when(j >= NBLK_NEURAL)
    def _():
        out_ref[...] = jnp.broadcast_to(csum, (B, TBLK))


def kernel(Z, atom_idx, obj_idx, w, base_idx):
    del atom_idx
    zflat = Z.reshape(B, N_OBJ * D).astype(jnp.bfloat16)
    wt = w.astype(jnp.bfloat16).T
    obj2 = obj_idx.astype(jnp.bfloat16).reshape(1, N_NEURAL)

    counts = _get_hist_kernel()(base_idx)

    return pl.pallas_call(
        _tc_body,
        grid=(NJ,),
        in_specs=[
            pl.BlockSpec((B, N_OBJ * D), lambda j: (0, 0)),
            pl.BlockSpec((B, TBLK), lambda j: (0, jnp.minimum(j, NBLK_NEURAL - 1))),
            pl.BlockSpec((1, TBLK), lambda j: (0, jnp.minimum(j, NBLK_NEURAL - 1))),
            pl.BlockSpec((2, TBLK), lambda j: (0, j)),
        ],
        out_specs=pl.BlockSpec((B, TBLK), lambda j: (0, j)),
        out_shape=jax.ShapeDtypeStruct((B, N_ATOMS), jnp.float32),
        scratch_shapes=[pltpu.VMEM((B, 128), jnp.float32)],
    )(zflat, wt, obj2, counts)


# speedup vs baseline: 2.1037x; 1.0478x over previous
import functools

import jax
import jax.numpy as jnp
from jax import lax
from jax.experimental import pallas as pl
from jax.experimental.pallas import tpu as pltpu
from jax.experimental.pallas import tpu_sc as plsc

B = 32
N_OBJ = 16
D = 32
N_NEURAL = 50000
N_ATOMS = 100000

TBLK = 8192
NJ = (N_ATOMS + TBLK - 1) // TBLK
NEURAL_HI = N_NEURAL + 2
NBLK_NEURAL = (NEURAL_HI + TBLK - 1) // TBLK
WPAD = NBLK_NEURAL * TBLK
NBINS = NJ * TBLK

N_BASE = 5000
PER_SUB = 160
CHUNK = 80
TAIL = N_BASE - 31 * PER_SUB
ZSPAN = NBINS // 16


def _hist_body(idx_hbm, out_hbm, idx_a, idx_b, idx_c, ones_v,
               buf_v, hist_sh):
    cid = lax.axis_index("c")
    sid = lax.axis_index("s")
    wid = cid * 16 + sid

    for i in range(CHUNK // 16):
        ones_v[pl.ds(i * 16, 16)] = jnp.full((16,), 1.0, jnp.float32)

    def _zero(i, carry):
        buf_v[pl.ds(i * 16, 16)] = jnp.zeros((16,), jnp.float32)
        return carry

    lax.fori_loop(0, ZSPAN // 16, _zero, 0)

    zbase = sid * ZSPAN
    pltpu.sync_copy(buf_v, hist_sh.at[pl.ds(zbase, ZSPAN)])
    plsc.subcore_barrier()

    base = wid * PER_SUB

    @pl.when(wid < 31)
    def _():
        pltpu.sync_copy(idx_hbm.at[pl.ds(base, CHUNK)], idx_a)
        pltpu.sync_copy(idx_hbm.at[pl.ds(base + CHUNK, CHUNK)], idx_b)
        pltpu.sync_copy(ones_v, hist_sh.at[idx_a], add=True)
        pltpu.sync_copy(ones_v, hist_sh.at[idx_b], add=True)

    @pl.when(wid == 31)
    def _():
        pltpu.sync_copy(idx_hbm.at[pl.ds(31 * PER_SUB, TAIL)], idx_c)
        pltpu.sync_copy(ones_v.at[pl.ds(0, TAIL)], hist_sh.at[idx_c], add=True)

    plsc.subcore_barrier()

    pltpu.sync_copy(hist_sh.at[pl.ds(zbase, ZSPAN)],
                    out_hbm.at[cid, pl.ds(zbase, ZSPAN)])


@functools.cache
def _get_hist_kernel():
    return pl.kernel(
        _hist_body,
        out_type=jax.ShapeDtypeStruct((2, NBINS), jnp.float32),
        mesh=plsc.VectorSubcoreMesh(core_axis_name="c", subcore_axis_name="s"),
        scratch_types=[
            pltpu.VMEM((CHUNK,), jnp.int32),
            pltpu.VMEM((CHUNK,), jnp.int32),
            pltpu.VMEM((TAIL,), jnp.int32),
            pltpu.VMEM((CHUNK,), jnp.float32),
            pltpu.VMEM((ZSPAN,), jnp.float32),
            pltpu.VMEM_SHARED((NBINS,), jnp.float32),
        ],
    )


def _tc_body(zf_ref, w_ref, obj_ref, cnt_ref, out_ref, carry_ref):
    j = pl.program_id(0)
    csum = cnt_ref[0:1, :] + cnt_ref[1:2, :]
    col = j * TBLK + lax.broadcasted_iota(jnp.int32, (1, TBLK), 1)

    @pl.when(j < NBLK_NEURAL)
    def _():
        wtb = w_ref[...]
        objr = obj_ref[...]
        objb = jnp.broadcast_to(objr, (B, TBLK))
        parts = [jnp.where(objb == jnp.bfloat16(o), wtb, jnp.bfloat16(0.0))
                 for o in range(N_OBJ)]
        u = jnp.concatenate(parts, axis=0)
        acc = jnp.dot(zf_ref[...], u, preferred_element_type=jnp.float32)
        vals = jax.nn.sigmoid(acc)
        prev = carry_ref[...]
        carry_ref[...] = vals[:, TBLK - 128:]
        sh = pltpu.roll(vals, 2, 1)
        rc = pltpu.roll(prev, 2, 1)
        head = jnp.where(lax.broadcasted_iota(jnp.int32, (1, 128), 1) < 2,
                         rc, sh[:, :128])
        vfull = jnp.concatenate([head, sh[:, 128:]], axis=1)
        neural = (col >= 2) & (col < NEURAL_HI)
        out = jnp.where(neural, vfull, 0.0) + csum
        out_ref[...] = jnp.where(col == 1, 1.0, out)

    @pl.when(j >= NBLK_NEURAL)
    def _():
        out_ref[...] = jnp.broadcast_to(csum, (B, TBLK))


def kernel(Z, atom_idx, obj_idx, w, base_idx):
    del atom_idx
    zflat = Z.reshape(B, N_OBJ * D).astype(jnp.bfloat16)
    wt = w.astype(jnp.bfloat16).T
    obj2 = obj_idx.astype(jnp.bfloat16).reshape(1, N_NEURAL)

    counts = _get_hist_kernel()(base_idx)

    return pl.pallas_call(
        _tc_body,
        grid=(NJ,),
        in_specs=[
            pl.BlockSpec((B, N_OBJ * D), lambda j: (0, 0)),
            pl.BlockSpec((B, TBLK), lambda j: (0, jnp.minimum(j, NBLK_NEURAL - 1))),
            pl.BlockSpec((1, TBLK), lambda j: (0, jnp.minimum(j, NBLK_NEURAL - 1))),
            pl.BlockSpec((2, TBLK), lambda j: (0, j)),
        ],
        out_specs=pl.BlockSpec((B, TBLK), lambda j: (0, j)),
        out_shape=jax.ShapeDtypeStruct((B, N_ATOMS), jnp.float32),
        scratch_shapes=[pltpu.VMEM((B, 128), jnp.float32)],
    )(zflat, wt, obj2, counts)


# speedup vs baseline: 2.1171x; 1.0064x over previous
import functools

import jax
import jax.numpy as jnp
from jax import lax
from jax.experimental import pallas as pl
from jax.experimental.pallas import tpu as pltpu
from jax.experimental.pallas import tpu_sc as plsc

B = 32
N_OBJ = 16
D = 32
N_NEURAL = 50000
N_ATOMS = 100000

TBLK = 16384
NJ = (N_ATOMS + TBLK - 1) // TBLK
NEURAL_HI = N_NEURAL + 2
NBLK_NEURAL = (NEURAL_HI + TBLK - 1) // TBLK
WPAD = NBLK_NEURAL * TBLK
NBINS = NJ * TBLK

N_BASE = 5000
PER_SUB = 160
CHUNK = 80
TAIL = N_BASE - 31 * PER_SUB
ZSPAN = NBINS // 16


def _hist_body(idx_hbm, out_hbm, idx_a, idx_b, idx_c, ones_v,
               buf_v, hist_sh):
    cid = lax.axis_index("c")
    sid = lax.axis_index("s")
    wid = cid * 16 + sid

    for i in range(CHUNK // 16):
        ones_v[pl.ds(i * 16, 16)] = jnp.full((16,), 1.0, jnp.float32)

    def _zero(i, carry):
        buf_v[pl.ds(i * 16, 16)] = jnp.zeros((16,), jnp.float32)
        return carry

    lax.fori_loop(0, ZSPAN // 16, _zero, 0)

    zbase = sid * ZSPAN
    pltpu.sync_copy(buf_v, hist_sh.at[pl.ds(zbase, ZSPAN)])
    plsc.subcore_barrier()

    base = wid * PER_SUB

    @pl.when(wid < 31)
    def _():
        pltpu.sync_copy(idx_hbm.at[pl.ds(base, CHUNK)], idx_a)
        pltpu.sync_copy(idx_hbm.at[pl.ds(base + CHUNK, CHUNK)], idx_b)
        pltpu.sync_copy(ones_v, hist_sh.at[idx_a], add=True)
        pltpu.sync_copy(ones_v, hist_sh.at[idx_b], add=True)

    @pl.when(wid == 31)
    def _():
        pltpu.sync_copy(idx_hbm.at[pl.ds(31 * PER_SUB, TAIL)], idx_c)
        pltpu.sync_copy(ones_v.at[pl.ds(0, TAIL)], hist_sh.at[idx_c], add=True)

    plsc.subcore_barrier()

    pltpu.sync_copy(hist_sh.at[pl.ds(zbase, ZSPAN)],
                    out_hbm.at[cid, pl.ds(zbase, ZSPAN)])


@functools.cache
def _get_hist_kernel():
    return pl.kernel(
        _hist_body,
        out_type=jax.ShapeDtypeStruct((2, NBINS), jnp.float32),
        mesh=plsc.VectorSubcoreMesh(core_axis_name="c", subcore_axis_name="s"),
        scratch_types=[
            pltpu.VMEM((CHUNK,), jnp.int32),
            pltpu.VMEM((CHUNK,), jnp.int32),
            pltpu.VMEM((TAIL,), jnp.int32),
            pltpu.VMEM((CHUNK,), jnp.float32),
            pltpu.VMEM((ZSPAN,), jnp.float32),
            pltpu.VMEM_SHARED((NBINS,), jnp.float32),
        ],
    )


def _tc_body(zf_ref, w_ref, obj_ref, cnt_ref, out_ref, carry_ref):
    j = pl.program_id(0)
    csum = cnt_ref[0:1, :] + cnt_ref[1:2, :]
    col = j * TBLK + lax.broadcasted_iota(jnp.int32, (1, TBLK), 1)

    @pl.when(j < NBLK_NEURAL)
    def _():
        wtb = w_ref[...]
        objr = obj_ref[...]
        objb = jnp.broadcast_to(objr, (B, TBLK))
        parts = [jnp.where(objb == jnp.bfloat16(o), wtb, jnp.bfloat16(0.0))
                 for o in range(N_OBJ)]
        u = jnp.concatenate(parts, axis=0)
        acc = jnp.dot(zf_ref[...], u, preferred_element_type=jnp.float32)
        vals = jax.nn.sigmoid(acc)
        prev = carry_ref[...]
        carry_ref[...] = vals[:, TBLK - 128:]
        sh = pltpu.roll(vals, 2, 1)
        rc = pltpu.roll(prev, 2, 1)
        head = jnp.where(lax.broadcasted_iota(jnp.int32, (1, 128), 1) < 2,
                         rc, sh[:, :128])
        vfull = jnp.concatenate([head, sh[:, 128:]], axis=1)
        neural = (col >= 2) & (col < NEURAL_HI)
        out = jnp.where(neural, vfull, 0.0) + csum
        out_ref[...] = jnp.where(col == 1, 1.0, out)

    @pl.when(j >= NBLK_NEURAL)
    def _():
        out_ref[...] = jnp.broadcast_to(csum, (B, TBLK))


def kernel(Z, atom_idx, obj_idx, w, base_idx):
    del atom_idx
    zflat = Z.reshape(B, N_OBJ * D).astype(jnp.bfloat16)
    wt = w.astype(jnp.bfloat16).T
    obj2 = obj_idx.astype(jnp.bfloat16).reshape(1, N_NEURAL)

    counts = _get_hist_kernel()(base_idx)

    return pl.pallas_call(
        _tc_body,
        grid=(NJ,),
        in_specs=[
            pl.BlockSpec((B, N_OBJ * D), lambda j: (0, 0)),
            pl.BlockSpec((B, TBLK), lambda j: (0, jnp.minimum(j, NBLK_NEURAL - 1))),
            pl.BlockSpec((1, TBLK), lambda j: (0, jnp.minimum(j, NBLK_NEURAL - 1))),
            pl.BlockSpec((2, TBLK), lambda j: (0, j)),
        ],
        out_specs=pl.BlockSpec((B, TBLK), lambda j: (0, j)),
        out_shape=jax.ShapeDtypeStruct((B, N_ATOMS), jnp.float32),
        scratch_shapes=[pltpu.VMEM((B, 128), jnp.float32)],
    )(zflat, wt, obj2, counts)
